# baseline (speedup 1.0000x reference)
import dataclasses
import functools

import jax
import jax.numpy as jnp
from jax import lax
from jax.experimental import pallas as pl
from jax.experimental.pallas import tpu as pltpu
from jax.experimental.pallas import tpu_sc as plsc

_B = 4096
_NU = 1000000
_L = 200
_D = 64
_NW = 32
_RPW = _B // _NW
_H = 104
_LP = 208
_NB = 4


def _bpr_sc(user, item_t, emb_cat, ub_flat, ib_flat):
    mesh = plsc.VectorSubcoreMesh(core_axis_name="c", subcore_axis_name="s")
    cp = pltpu.CompilerParams()
    if "needs_layout_passes" in pltpu.CompilerParams.__dataclass_fields__:
        cp = dataclasses.replace(cp, needs_layout_passes=False)
    if "use_tc_tiling_on_sc" in pltpu.CompilerParams.__dataclass_fields__:
        cp = dataclasses.replace(cp, use_tc_tiling_on_sc=False)

    @functools.partial(
        pl.kernel,
        out_type=jax.ShapeDtypeStruct((_B, _L), jnp.float32),
        mesh=mesh,
        compiler_params=cp,
        scratch_types=[
            pltpu.VMEM((_LP, _RPW), jnp.int32),
            pltpu.VMEM((_RPW,), jnp.int32),
            pltpu.VMEM((_RPW,), jnp.int32),
            pltpu.VMEM((_RPW, _D), jnp.bfloat16),
            pltpu.VMEM((_RPW + 16,), jnp.float32),
            pltpu.VMEM((_NB, _LP + 16), jnp.int32),
            pltpu.VMEM((_NB, _LP + 16), jnp.int32),
            pltpu.VMEM((_NB, _LP, _D), jnp.bfloat16),
            pltpu.VMEM((_NB, _LP), jnp.float32),
            pltpu.VMEM((_NB, _LP + 16), jnp.float32),
            pltpu.SemaphoreType.DMA,
            [pltpu.SemaphoreType.DMA] * _NB,
            [pltpu.SemaphoreType.DMA] * _NB,
        ],
    )
    def k(user_h, item_h, emb_h, ubias_h, ibias_h, out_h,
          slab, uidx, uidx2, urows, ubias, iidx, iidxb, irows, ibias, obuf,
          sem_u, sem_g, sem_o):
        wid = lax.axis_index("s") * 2 + lax.axis_index("c")
        base = wid * _RPW

        lanes = lax.iota(jnp.int32, 16)
        last_mask = lanes == 15
        fzeros16 = jnp.zeros((16,), jnp.float32)

        pltpu.sync_copy(user_h.at[pl.ds(base, _RPW)], uidx)
        for c in range(_RPW // 16):
            uslc = pl.ds(c * 16, 16)
            uidx2[uslc] = uidx[uslc] * 2
        cu1 = pltpu.async_copy(emb_h.at[uidx2], urows, sem_u)
        cu2 = pltpu.async_copy(ubias_h.at[uidx], ubias.at[pl.ds(0, _RPW)],
                               sem_u)
        pltpu.sync_copy(item_h.at[pl.ds(0, _L), pl.ds(base, _RPW)],
                        slab.at[pl.ds(0, _L), pl.ds(0, _RPW)])

        def assemble_idx(r_rel, b):
            rsplat = jnp.full((16,), r_rel, jnp.int32)
            for c in range(_LP // 16):
                l16 = lanes + c * 16
                if (c + 1) * 16 > _L:
                    l16 = jnp.minimum(l16, _L - 1)
                v = plsc.load_gather(slab, [l16, rsplat])
                iidx[b, pl.ds(c * 16, 16)] = v * 2 + 1
                iidxb[b, pl.ds(c * 16, 16)] = v

        def fire(b, sem):
            pltpu.async_copy(emb_h.at[iidx.at[b, pl.ds(0, _H)]],
                             irows.at[b, pl.ds(0, _H)], sem)
            pltpu.async_copy(emb_h.at[iidx.at[b, pl.ds(_H, _H)]],
                             irows.at[b, pl.ds(_H, _H)], sem)
            pltpu.async_copy(ibias_h.at[iidxb.at[b, pl.ds(0, _H)]],
                             ibias.at[b, pl.ds(0, _H)], sem)
            pltpu.async_copy(ibias_h.at[iidxb.at[b, pl.ds(_H, _H)]],
                             ibias.at[b, pl.ds(_H, _H)], sem)

        def wait_gather(b, sem):
            pltpu.make_async_copy(emb_h.at[iidx.at[b, pl.ds(0, _H)]],
                                  irows.at[b, pl.ds(0, _H)], sem).wait()
            pltpu.make_async_copy(emb_h.at[iidx.at[b, pl.ds(_H, _H)]],
                                  irows.at[b, pl.ds(_H, _H)], sem).wait()
            pltpu.make_async_copy(ibias_h.at[iidxb.at[b, pl.ds(0, _H)]],
                                  ibias.at[b, pl.ds(0, _H)], sem).wait()
            pltpu.make_async_copy(ibias_h.at[iidxb.at[b, pl.ds(_H, _H)]],
                                  ibias.at[b, pl.ds(_H, _H)], sem).wait()

        def wait_out(b, sem):
            pltpu.make_async_copy(obuf.at[b, pl.ds(0, _L)],
                                  out_h.at[base, pl.ds(0, _L)], sem).wait()

        def compute(r_rel, b):
            fmt = plsc.PackFormat.INTERLEAVED
            u0, u1 = plsc.unpack(urows[r_rel, pl.ds(0, 32)], format=fmt)
            u2, u3 = plsc.unpack(urows[r_rel, pl.ds(32, 32)], format=fmt)
            ubv = ubias[pl.ds(r_rel, 16)]
            ub_vec0 = jnp.where(lanes == 0, ubv, fzeros16)

            @pl.loop(0, _LP // 16)
            def _(c):
                l0 = c * 16
                for j in range(16):
                    l = l0 + j
                    i0, i1 = plsc.unpack(irows[b, l, pl.ds(0, 32)],
                                         format=fmt)
                    i2, i3 = plsc.unpack(irows[b, l, pl.ds(32, 32)],
                                         format=fmt)
                    p = (ub_vec0 + i0 * u0 + i1 * u1 + i2 * u2 + i3 * u3)
                    cs = plsc.cumsum(p)
                    plsc.store_compressed(obuf.at[b, pl.ds(l, 16)], cs,
                                          mask=last_mask)
                obuf[b, pl.ds(l0, 16)] = (obuf[b, pl.ds(l0, 16)]
                                          + ibias[b, pl.ds(l0, 16)])

        cu1.wait()
        cu2.wait()
        for b in range(_NB):
            assemble_idx(b, b)
            fire(b, sem_g[b])

        @pl.loop(0, _RPW, step=_NB)
        def _(kk):
            for b in range(_NB):
                r = kk + b
                wait_gather(b, sem_g[b])

                @pl.when(kk >= _NB)
                def _():
                    wait_out(b, sem_o[b])

                compute(r, b)
                pltpu.async_copy(obuf.at[b, pl.ds(0, _L)],
                                 out_h.at[base + r, pl.ds(0, _L)], sem_o[b])

                @pl.when(r + _NB < _RPW)
                def _():
                    assemble_idx(r + _NB, b)
                    fire(b, sem_g[b])

        for b in range(_NB):
            wait_out(b, sem_o[b])

    return k(user, item_t, emb_cat, ub_flat, ib_flat)


def kernel(user, item, u_emb, i_emb, user_bias, item_bias):
    emb_cat = (jnp.concatenate([u_emb, i_emb], axis=1)
               .astype(jnp.bfloat16).reshape(2 * _NU, _D))
    return _bpr_sc(user, item.T, emb_cat,
                   user_bias.reshape(-1), item_bias.reshape(-1))


# speedup vs baseline: 1.6843x; 1.6843x over previous
import dataclasses
import functools

import jax
import jax.numpy as jnp
from jax import lax
from jax.experimental import pallas as pl
from jax.experimental.pallas import tpu as pltpu
from jax.experimental.pallas import tpu_sc as plsc

_B = 4096
_NU = 1000000
_L = 200
_D = 64
_NW = 32
_RPW = _B // _NW
_H = 104
_LP = 208
_NB = 4


def _bpr_sc(user, item_t, emb_cat, ub_flat, ib_flat):
    mesh = plsc.VectorSubcoreMesh(core_axis_name="c", subcore_axis_name="s")
    cp = pltpu.CompilerParams()
    if "needs_layout_passes" in pltpu.CompilerParams.__dataclass_fields__:
        cp = dataclasses.replace(cp, needs_layout_passes=False)
    if "use_tc_tiling_on_sc" in pltpu.CompilerParams.__dataclass_fields__:
        cp = dataclasses.replace(cp, use_tc_tiling_on_sc=False)

    @functools.partial(
        pl.kernel,
        out_type=jax.ShapeDtypeStruct((_B, _L), jnp.float32),
        mesh=mesh,
        compiler_params=cp,
        scratch_types=[
            pltpu.VMEM((_LP, _RPW), jnp.int32),
            pltpu.VMEM((_RPW,), jnp.int32),
            pltpu.VMEM((_RPW,), jnp.int32),
            pltpu.VMEM((_RPW, _D), jnp.float32),
            pltpu.VMEM((_RPW + 16,), jnp.float32),
            pltpu.VMEM((_NB, _LP + 16), jnp.int32),
            pltpu.VMEM((_NB, _LP + 16), jnp.int32),
            pltpu.VMEM((_NB, _LP, _D), jnp.float32),
            pltpu.VMEM((_NB, _LP), jnp.float32),
            pltpu.VMEM((_NB, _LP + 16), jnp.float32),
            pltpu.SemaphoreType.DMA,
            [pltpu.SemaphoreType.DMA] * _NB,
            [pltpu.SemaphoreType.DMA] * _NB,
        ],
    )
    def k(user_h, item_h, emb_h, ubias_h, ibias_h, out_h,
          slab, uidx, uidx2, urows, ubias, iidx, iidxb, irows,
          ibias, obuf, sem_u, sem_g, sem_o):
        wid = lax.axis_index("s") * 2 + lax.axis_index("c")
        base = wid * _RPW

        lanes = lax.iota(jnp.int32, 16)
        last_mask = lanes == 15
        fzeros16 = jnp.zeros((16,), jnp.float32)

        pltpu.sync_copy(user_h.at[pl.ds(base, _RPW)], uidx)
        for c in range(_RPW // 16):
            uslc = pl.ds(c * 16, 16)
            uidx2[uslc] = uidx[uslc] * 2
        cu1 = pltpu.async_copy(emb_h.at[uidx2], urows, sem_u)
        cu2 = pltpu.async_copy(ubias_h.at[uidx], ubias.at[pl.ds(0, _RPW)],
                               sem_u)
        pltpu.sync_copy(item_h.at[pl.ds(0, _L), pl.ds(base, _RPW)],
                        slab.at[pl.ds(0, _L), pl.ds(0, _RPW)])


        def assemble_idx(r_rel, b):
            rsplat = jnp.full((16,), r_rel, jnp.int32)
            for c in range(_LP // 16):
                l16 = lanes + c * 16
                if (c + 1) * 16 > _L:
                    l16 = jnp.minimum(l16, _L - 1)
                v = plsc.load_gather(slab, [l16, rsplat])
                iidx[b, pl.ds(c * 16, 16)] = v * 2 + 1
                iidxb[b, pl.ds(c * 16, 16)] = v

        _H2 = _L - _H

        def fire(b, sem):
            pltpu.async_copy(emb_h.at[iidx.at[b, pl.ds(0, _H)]],
                             irows.at[b, pl.ds(0, _H)], sem)
            pltpu.async_copy(emb_h.at[iidx.at[b, pl.ds(_H, _H2)]],
                             irows.at[b, pl.ds(_H, _H2)], sem)
            pltpu.async_copy(ibias_h.at[iidxb.at[b, pl.ds(0, _H)]],
                             ibias.at[b, pl.ds(0, _H)], sem)
            pltpu.async_copy(ibias_h.at[iidxb.at[b, pl.ds(_H, _H2)]],
                             ibias.at[b, pl.ds(_H, _H2)], sem)

        def wait_gather(b, sem):
            pltpu.make_async_copy(emb_h.at[iidx.at[b, pl.ds(0, _H)]],
                                  irows.at[b, pl.ds(0, _H)], sem).wait()
            pltpu.make_async_copy(emb_h.at[iidx.at[b, pl.ds(_H, _H2)]],
                                  irows.at[b, pl.ds(_H, _H2)], sem).wait()
            pltpu.make_async_copy(ibias_h.at[iidxb.at[b, pl.ds(0, _H)]],
                                  ibias.at[b, pl.ds(0, _H)], sem).wait()
            pltpu.make_async_copy(ibias_h.at[iidxb.at[b, pl.ds(_H, _H2)]],
                                  ibias.at[b, pl.ds(_H, _H2)], sem).wait()

        def wait_out(b, sem):
            pltpu.make_async_copy(obuf.at[b, pl.ds(0, _L)],
                                  out_h.at[base, pl.ds(0, _L)], sem).wait()

        def compute(r_rel, b):
            u0 = urows[r_rel, pl.ds(0, 16)]
            u1 = urows[r_rel, pl.ds(16, 16)]
            u2 = urows[r_rel, pl.ds(32, 16)]
            u3 = urows[r_rel, pl.ds(48, 16)]
            ubv = ubias[pl.ds(r_rel, 16)]
            ub_vec0 = jnp.where(lanes == 0, ubv, fzeros16)

            @pl.loop(0, _LP // 16)
            def _(c):
                l0 = c * 16
                for j in range(16):
                    l = l0 + j
                    p = (ub_vec0
                         + irows[b, l, pl.ds(0, 16)] * u0
                         + irows[b, l, pl.ds(16, 16)] * u1
                         + irows[b, l, pl.ds(32, 16)] * u2
                         + irows[b, l, pl.ds(48, 16)] * u3)
                    cs = plsc.cumsum(p)
                    plsc.store_compressed(obuf.at[b, pl.ds(l, 16)], cs,
                                          mask=last_mask)
                obuf[b, pl.ds(l0, 16)] = (obuf[b, pl.ds(l0, 16)]
                                          + ibias[b, pl.ds(l0, 16)])

        cu1.wait()
        cu2.wait()
        for b in range(_NB):
            assemble_idx(b, b)
            fire(b, sem_g[b])

        @pl.loop(0, _RPW, step=_NB)
        def _(kk):
            for b in range(_NB):
                r = kk + b
                wait_gather(b, sem_g[b])

                @pl.when(kk >= _NB)
                def _():
                    wait_out(b, sem_o[b])

                compute(r, b)
                pltpu.async_copy(obuf.at[b, pl.ds(0, _L)],
                                 out_h.at[base + r, pl.ds(0, _L)], sem_o[b])

                @pl.when(r + _NB < _RPW)
                def _():
                    assemble_idx(r + _NB, b)
                    fire(b, sem_g[b])

        for b in range(_NB):
            wait_out(b, sem_o[b])

    return k(user, item_t, emb_cat, ub_flat, ib_flat)


def kernel(user, item, u_emb, i_emb, user_bias, item_bias):
    emb_cat = jnp.concatenate([u_emb, i_emb], axis=1).reshape(2 * _NU, _D)
    return _bpr_sc(user, item.T, emb_cat,
                   user_bias.reshape(-1), item_bias.reshape(-1))
